# Initial kernel scaffold; baseline (speedup 1.0000x reference)
import jax
import jax.numpy as jnp
from jax.experimental import pallas as pl


def kernel(x, adj, W1, W2, W3):
    raise NotImplementedError("write your pallas kernel here")


# speedup vs baseline: 1.4287x; 1.4287x over previous
import functools

import jax
import jax.numpy as jnp
from jax.experimental import pallas as pl
from jax.experimental.pallas import tpu as pltpu


def _gnp_kernel(x_ref, adj_ref, w1_ref, wcat_ref, mu_ref, lv_ref,
                s_ref, h1_ref, b_ref, *, blk, zdim):
    p = pl.program_id(0)
    i = pl.program_id(1)

    @pl.when(jnp.logical_and(p == 0, i == 0))
    def _init():
        s_ref[:] = jnp.dot(x_ref[:], w1_ref[:],
                           preferred_element_type=jnp.float32)

    @pl.when(p == 0)
    def _phase0():
        h1_ref[pl.ds(i * blk, blk), :] = jnp.maximum(
            jnp.dot(adj_ref[:], s_ref[:],
                    preferred_element_type=jnp.float32), 0.0)

    @pl.when(p == 1)
    def _phase1():
        @pl.when(i == 0)
        def _mid():
            b_ref[:] = jnp.dot(h1_ref[:], wcat_ref[:],
                               preferred_element_type=jnp.float32)

        e = jnp.dot(adj_ref[:], b_ref[:],
                    preferred_element_type=jnp.float32)
        mu_ref[:] = e[:, :zdim]
        lv_ref[:] = e[:, zdim:]


def kernel(x, adj, W1, W2, W3):
    n, d = x.shape
    h_dim = W1.shape[1]
    z_dim = W2.shape[1]

    wcat = jnp.concatenate([W2, W3], axis=1)

    blk = 200
    grid = n // blk

    mu, lv = pl.pallas_call(
        functools.partial(_gnp_kernel, blk=blk, zdim=z_dim),
        grid=(2, grid),
        in_specs=[
            pl.BlockSpec((n, d), lambda p, i: (0, 0)),
            pl.BlockSpec((blk, n), lambda p, i: (i, 0)),
            pl.BlockSpec((d, h_dim), lambda p, i: (0, 0)),
            pl.BlockSpec((h_dim, 2 * z_dim), lambda p, i: (0, 0)),
        ],
        out_specs=[
            pl.BlockSpec((blk, z_dim), lambda p, i: (i, 0)),
            pl.BlockSpec((blk, z_dim), lambda p, i: (i, 0)),
        ],
        out_shape=[
            jax.ShapeDtypeStruct((n, z_dim), jnp.float32),
            jax.ShapeDtypeStruct((n, z_dim), jnp.float32),
        ],
        scratch_shapes=[
            pltpu.VMEM((n, h_dim), jnp.float32),
            pltpu.VMEM((n, h_dim), jnp.float32),
            pltpu.VMEM((n, 2 * z_dim), jnp.float32),
        ],
    )(x, adj, W1, wcat)

    eye = jnp.eye(z_dim, dtype=jnp.float32)
    mu = jnp.dot(mu, eye, precision=jax.lax.Precision.HIGHEST)
    lv = jnp.dot(lv, eye, precision=jax.lax.Precision.HIGHEST)
    std = jnp.exp(lv)
    z_logvar = jnp.log(jnp.mean(std))
    z_mu = jnp.mean(mu)
    return (z_mu, z_logvar)


# speedup vs baseline: 1.4553x; 1.0186x over previous
import functools

import jax
import jax.numpy as jnp
from jax.experimental import pallas as pl
from jax.experimental.pallas import tpu as pltpu


def _gnp_kernel(x_ref, adj_ref, w1_ref, wcat_ref, mu_ref, lv_ref,
                s_ref, h1_ref, b_ref, *, blk, zdim):
    p = pl.program_id(0)
    i = pl.program_id(1)

    @pl.when(jnp.logical_and(p == 0, i == 0))
    def _init():
        s_ref[:] = jnp.dot(x_ref[:], w1_ref[:],
                           preferred_element_type=jnp.float32)

    @pl.when(p == 0)
    def _phase0():
        h1_ref[pl.ds(i * blk, blk), :] = jnp.maximum(
            jnp.dot(adj_ref[:], s_ref[:],
                    preferred_element_type=jnp.float32), 0.0)

    @pl.when(p == 1)
    def _phase1():
        @pl.when(i == 0)
        def _mid():
            b_ref[:] = jnp.dot(h1_ref[:], wcat_ref[:],
                               preferred_element_type=jnp.float32)

        e = jnp.dot(adj_ref[:], b_ref[:],
                    preferred_element_type=jnp.float32)
        mu_ref[:] = e[:, :zdim]
        lv_ref[:] = e[:, zdim:]


def kernel(x, adj, W1, W2, W3):
    n, d = x.shape
    h_dim = W1.shape[1]
    z_dim = W2.shape[1]

    wcat = jnp.concatenate([W2, W3], axis=1)

    blk = 200
    grid = n // blk

    mu, lv = pl.pallas_call(
        functools.partial(_gnp_kernel, blk=blk, zdim=z_dim),
        grid=(2, grid),
        in_specs=[
            pl.BlockSpec((n, d), lambda p, i: (0, 0)),
            pl.BlockSpec((blk, n), lambda p, i: (i, 0)),
            pl.BlockSpec((d, h_dim), lambda p, i: (0, 0)),
            pl.BlockSpec((h_dim, 2 * z_dim), lambda p, i: (0, 0)),
        ],
        out_specs=[
            pl.BlockSpec((blk, z_dim), lambda p, i: (i, 0)),
            pl.BlockSpec((blk, z_dim), lambda p, i: (i, 0)),
        ],
        out_shape=[
            jax.ShapeDtypeStruct((n, z_dim), jnp.float32),
            jax.ShapeDtypeStruct((n, z_dim), jnp.float32),
        ],
        scratch_shapes=[
            pltpu.VMEM((n, h_dim), jnp.float32),
            pltpu.VMEM((n, h_dim), jnp.float32),
            pltpu.VMEM((n, 2 * z_dim), jnp.float32),
        ],
    )(x, adj, W1, wcat)

    std = jnp.exp(lv)
    z_logvar = jnp.log(jnp.mean(std))
    z_mu = jnp.mean(mu)
    return (z_mu, z_logvar)
